# baseline (speedup 1.0000x reference)
import functools

import jax
import jax.numpy as jnp
import numpy as np
from jax import lax
from jax.experimental import pallas as pl
from jax.experimental.pallas import tpu as pltpu
from jax.experimental.pallas import tpu_sc as plsc

N = 10000
E = 320000
HEADS = 8
CDIM = 16
NH = HEADS * CDIM
NLABEL = 40

_SS = np.zeros((NH, 16), np.float32)
for _h in range(HEADS):
    _SS[_h * CDIM:(_h + 1) * CDIM, _h] = 1.0
    _SS[_h * CDIM:(_h + 1) * CDIM, 8 + _h] = 1.0
_D = np.zeros((16, NH), np.float32)
for _h in range(HEADS):
    _D[_h, _h * CDIM:(_h + 1) * CDIM] = 1.0

_RB = 2000
_GRID = N // _RB

_NC = 2
_NS = 16
_NW = _NC * _NS
_EPW = E // _NW
_K = 16
_NCH = _EPW // _K
_NPAIR = _NCH // 2
_IR = (_EPW + 1279) // 1280 * 10
_NACC = 10112
_NWR = _NACC // 8
_TROWS = _NACC + _NWR
_RPT = 712
_RPT_LAST = _TROWS - 15 * _RPT


def _make_sc_edge_pass():
    mesh = plsc.VectorSubcoreMesh(
        core_axis_name="c", subcore_axis_name="s",
        num_cores=_NC, num_subcores=_NS)

    @functools.partial(
        pl.kernel,
        out_type=jax.ShapeDtypeStruct((_NC * _TROWS, NH), jnp.float32),
        mesh=mesh,
        scratch_types=[
            pltpu.VMEM((_IR, 128), jnp.int32),
            pltpu.VMEM((_K, 2 * NH), jnp.float32),
            pltpu.VMEM((_K, 2 * NH), jnp.float32),
            pltpu.VMEM((_K, 2 * NH), jnp.float32),
            pltpu.VMEM((_K, 2 * NH), jnp.float32),
            pltpu.VMEM((_K, NH), jnp.float32),
            pltpu.VMEM((_K, NH), jnp.float32),
            pltpu.VMEM((_K, NH), jnp.float32),
            pltpu.VMEM((_K, NH), jnp.float32),
            pltpu.VMEM((2 * _K, NH), jnp.float32),
            pltpu.VMEM((2 * _K,), jnp.int32),
            pltpu.VMEM((16,), jnp.int32),
            pltpu.VMEM_SHARED((_TROWS, NH), jnp.float32),
            pltpu.SemaphoreType.DMA,
            pltpu.SemaphoreType.DMA,
            pltpu.SemaphoreType.DMA,
            pltpu.SemaphoreType.DMA,
            pltpu.SemaphoreType.DMA,
        ],
    )
    def sc_edge_pass(hfts_hbm, td_hbm, sd_hbm, zer_hbm, out_hbm,
                     sd_i, hb0, hb1, hb2, hb3, tb0, tb1, tb2, tb3,
                     cb, ci, offp, acc_sh,
                     gsem0, gsem1, gsem2, gsem3, ssem):
        cid = lax.axis_index("c")
        sid = lax.axis_index("s")
        wid = sid * _NC + cid

        @pl.when(sid < _NS - 1)
        def _():
            pltpu.sync_copy(zer_hbm, acc_sh.at[pl.ds(sid * _RPT, _RPT)])

        @pl.when(sid == _NS - 1)
        def _():
            pltpu.sync_copy(
                zer_hbm.at[pl.ds(0, _RPT_LAST)],
                acc_sh.at[pl.ds((_NS - 1) * _RPT, _RPT_LAST)])

        plsc.subcore_barrier()

        pltpu.sync_copy(sd_hbm.at[wid], sd_i)

        hb = (hb0, hb1, hb2, hb3)
        tb = (tb0, tb1, tb2, tb3)
        gsem = (gsem0, gsem1, gsem2, gsem3)
        zeros16 = jnp.zeros((16,), jnp.float32)

        for r in range(_K, 2 * _K):
            for h in range(HEADS):
                cb[r, h * 16:(h + 1) * 16] = zeros16
        offp[...] = jnp.zeros((16,), jnp.int32)

        def idx16(j):
            sd = sd_i[lax.shift_right_logical(j, 3),
                      pl.ds((j & 7) * 16, 16)]
            return sd & 0xFFFF, lax.shift_right_logical(sd, 16)

        def issue(j, slot):
            sv, dv = idx16(j)
            pltpu.async_copy(hfts_hbm.at[sv], hb[slot], gsem[slot])
            pltpu.async_copy(td_hbm.at[dv], tb[slot], gsem[slot])

        def wait2(slot):
            pltpu.make_async_copy(
                hfts_hbm.at[pl.ds(0, _K)], hb[slot], gsem[slot]).wait()
            pltpu.make_async_copy(
                td_hbm.at[pl.ds(0, _K)], tb[slot], gsem[slot]).wait()

        def s_drain():
            pltpu.make_async_copy(
                zer_hbm.at[pl.ds(0, 2 * _K)], cb, ssem).wait()

        def process(j, slot):
            s_drain()
            bh = hb[slot]
            bt = tb[slot]
            _, dv = idx16(j)
            ci[pl.ds(0, 16)] = dv
            ci[pl.ds(16, 16)] = _NACC + lax.shift_right_logical(dv, 3)
            offs = (dv & 7) * 16
            ov = offp[...]
            for j2 in range(16):
                w = bh[j2, 128:144] + bt[j2, 0:16]
                w = jnp.exp(jnp.where(w >= 0.0, w, 0.2 * w))
                for h in range(HEADS):
                    cb[j2, h * 16:(h + 1) * 16] = (
                        bh[j2, h * 16:(h + 1) * 16] * w[h])
                cb[_K + j2, pl.ds(ov[j2], 16)] = zeros16
                cb[_K + j2, pl.ds(offs[j2], 16)] = w
            offp[...] = offs
            pltpu.async_copy(cb, acc_sh.at[ci], ssem, add=True)

        ci[pl.ds(0, 16)] = jnp.full((16,), _NACC - 8, jnp.int32)
        ci[pl.ds(16, 16)] = jnp.full((16,), _TROWS - 8, jnp.int32)
        pltpu.async_copy(cb, acc_sh.at[ci], ssem, add=True)

        issue(0, 0)
        issue(1, 1)
        issue(2, 2)

        def quad_body(q, carry):
            j0 = q * 4
            for c in range(4):
                wait2(c)
                issue(j0 + c + 3, (c + 3) % 4)
                process(j0 + c, c)
            return carry

        lax.fori_loop(0, (_NCH - 1) // 4, quad_body, 0)
        wait2(0)
        process(_NCH - 1, 0)
        wait2(1)
        wait2(2)
        s_drain()

        plsc.subcore_barrier()

        @pl.when(sid < _NS - 1)
        def _():
            pltpu.sync_copy(
                acc_sh.at[pl.ds(sid * _RPT, _RPT)],
                out_hbm.at[pl.ds(cid * _TROWS + sid * _RPT, _RPT)])

        @pl.when(sid == _NS - 1)
        def _():
            pltpu.sync_copy(
                acc_sh.at[pl.ds((_NS - 1) * _RPT, _RPT_LAST)],
                out_hbm.at[pl.ds(cid * _TROWS + (_NS - 1) * _RPT,
                                 _RPT_LAST)])

    return sc_edge_pass


_sc_edge_cached = None


def _sc_edge_pass(hfts, td, sd, zer):
    global _sc_edge_cached
    if _sc_edge_cached is None:
        _sc_edge_cached = _make_sc_edge_pass()
    return _sc_edge_cached(hfts, td, sd, zer)


def _wblock(shape):
    nd = len(shape)
    return pl.BlockSpec(shape, lambda i: (0,) * nd)


def _rblock(lanes):
    return pl.BlockSpec((_RB, lanes), lambda i: (i, 0))


def _layer_tail(hf, aS, aD, Dm):
    ts = jnp.dot(hf, aS, preferred_element_type=jnp.float32)
    td = jnp.dot(hf, aD, preferred_element_type=jnp.float32)
    es = ts[:, :16] + td[:, :16]
    ws = jnp.exp(jnp.where(es >= 0.0, es, 0.2 * es))
    sm = hf * jnp.dot(ws, Dm, preferred_element_type=jnp.float32)
    hfts = jnp.concatenate([hf, ts], axis=-1)
    return hfts, td, sm, ws


_TC_PRE_IN = [
    _rblock(128), _wblock((128, 128)), _wblock((128, 128)),
    _wblock((128, 128)), _wblock((16, 128))]
_TC5_OUT = [_rblock(256), _rblock(128), _rblock(128), _rblock(16)]
_TC5_SHAPE = [jax.ShapeDtypeStruct((N, 256), jnp.float32),
              jax.ShapeDtypeStruct((N, 128), jnp.float32),
              jax.ShapeDtypeStruct((N, 128), jnp.float32),
              jax.ShapeDtypeStruct((N, 16), jnp.float32)]


def _tc_pre_body(x_ref, w_ref, aS_ref, aD_ref, D_ref,
                 hfts_ref, td_ref, sm_ref, sw_ref):
    hf = jnp.dot(x_ref[...], w_ref[...], preferred_element_type=jnp.float32)
    hfts, td, sm, ws = _layer_tail(hf, aS_ref[...], aD_ref[...], D_ref[...])
    hfts_ref[...] = hfts
    td_ref[...] = td
    sm_ref[...] = sm
    sw_ref[...] = ws


_tc_pre = pl.pallas_call(
    _tc_pre_body,
    grid=(_GRID,),
    in_specs=_TC_PRE_IN,
    out_specs=_TC5_OUT,
    out_shape=_TC5_SHAPE,
)


def _combine(a0m_ref, a1m_ref, a0w_ref, a1w_ref, sm_ref, sw_ref, b_ref, Dm):
    out = a0m_ref[...] + a1m_ref[...] + sm_ref[...]
    den = a0w_ref[...] + a1w_ref[...] + sw_ref[...]
    dend = jnp.dot(den, Dm, preferred_element_type=jnp.float32) + 1e-16
    z = out / dend + b_ref[...]
    return jnp.where(z > 0.0, z, jnp.exp(jnp.minimum(z, 0.0)) - 1.0)


def _tc_mid_body(a0m_ref, a1m_ref, a0w_ref, a1w_ref, sm0_ref, sw0_ref,
                 b_ref, w_ref, aS_ref, aD_ref, D_ref,
                 hfts_ref, td_ref, sm_ref, sw_ref):
    x1 = _combine(a0m_ref, a1m_ref, a0w_ref, a1w_ref, sm0_ref, sw0_ref,
                  b_ref, D_ref[...])
    hf = jnp.dot(x1, w_ref[...], preferred_element_type=jnp.float32)
    hfts, td, sm, ws = _layer_tail(hf, aS_ref[...], aD_ref[...], D_ref[...])
    hfts_ref[...] = hfts
    td_ref[...] = td
    sm_ref[...] = sm
    sw_ref[...] = ws


def _acc_blocks():
    return [_rblock(128), _rblock(128), _rblock(16), _rblock(16)]


_TC_MID_IN = _acc_blocks() + [
    _rblock(128), _rblock(16), _wblock((1, 128)), _wblock((128, 128)),
    _wblock((128, 128)), _wblock((128, 128)), _wblock((16, 128))]
_TC_FIN_IN = _acc_blocks() + [
    _rblock(128), _rblock(16), _wblock((1, 128)), _wblock((128, 256)),
    _wblock((1, 256)), _wblock((256, 128)), _wblock((1, 128)),
    _wblock((16, 128))]

_tc_mid = pl.pallas_call(
    _tc_mid_body,
    grid=(_GRID,),
    in_specs=_TC_MID_IN,
    out_specs=_TC5_OUT,
    out_shape=_TC5_SHAPE,
)


def _tc_fin_body(a0m_ref, a1m_ref, a0w_ref, a1w_ref, sm1_ref, sw1_ref,
                 b_ref, wf1_ref, bf1_ref, wf2_ref, bf2_ref, D_ref, out_ref):
    x2 = _combine(a0m_ref, a1m_ref, a0w_ref, a1w_ref, sm1_ref, sw1_ref,
                  b_ref, D_ref[...])
    z = jnp.dot(x2, wf1_ref[...], preferred_element_type=jnp.float32)
    z = z + bf1_ref[...]
    z = jnp.where(z > 0.0, z, jnp.exp(jnp.minimum(z, 0.0)) - 1.0)
    z = jnp.dot(z, wf2_ref[...], preferred_element_type=jnp.float32)
    z = z + bf2_ref[...]
    m = jnp.max(z, axis=-1, keepdims=True)
    lse = jnp.log(jnp.sum(jnp.exp(z - m), axis=-1, keepdims=True)) + m
    out_ref[...] = z - lse


_tc_fin = pl.pallas_call(
    _tc_fin_body,
    grid=(_GRID,),
    in_specs=_TC_FIN_IN,
    out_specs=_rblock(128),
    out_shape=jax.ShapeDtypeStruct((N, 128), jnp.float32),
)


def kernel(x, adj, W0, att_src0, att_dst0, b0, W1, att_src1, att_dst1, b1,
           Wf1, bf1, Wf2, bf2):
    SS = jnp.asarray(_SS)
    Dm = jnp.asarray(_D)
    pad = ((0, 0), (0, NH - 16))
    aS0 = jnp.pad(att_src0.reshape(NH)[:, None] * SS, pad)
    aD0 = jnp.pad(att_dst0.reshape(NH)[:, None] * SS, pad)
    aS1 = jnp.pad(att_src1.reshape(NH)[:, None] * SS, pad)
    aD1 = jnp.pad(att_dst1.reshape(NH)[:, None] * SS, pad)
    idx_pad = ((0, 0), (0, _IR * 128 - _EPW))
    sdp = (adj[1] << 16) | adj[0]
    sd = jnp.pad(sdp.reshape(_NW, _EPW), idx_pad).reshape(_NW, _IR, 128)
    zer = jnp.zeros((_RPT, NH), jnp.float32)
    wf2p = jnp.concatenate(
        [Wf2, jnp.zeros((2 * NH, 128 - NLABEL), jnp.float32)], axis=1)
    bf2p = jnp.concatenate(
        [bf2, jnp.full((128 - NLABEL,), -1e30, jnp.float32)]).reshape(1, 128)

    def acc_slices(acc):
        m0 = acc[:N]
        w0 = acc[_NACC:_TROWS].reshape(_NACC, 16)[:N]
        m1 = acc[_TROWS:_TROWS + N]
        w1 = acc[_TROWS + _NACC:2 * _TROWS].reshape(_NACC, 16)[:N]
        return m0, m1, w0, w1

    hfts0, td0, sm0, sw0 = _tc_pre(x, W0, aS0, aD0, Dm)
    acc0 = _sc_edge_pass(hfts0, td0, sd, zer)
    hfts1, td1, sm1, sw1 = _tc_mid(
        *acc_slices(acc0), sm0, sw0, b0.reshape(1, NH), W1, aS1, aD1, Dm)
    acc1 = _sc_edge_pass(hfts1, td1, sd, zer)
    logp = _tc_fin(
        *acc_slices(acc1), sm1, sw1, b1.reshape(1, NH), Wf1,
        bf1.reshape(1, 2 * NH), wf2p, bf2p, Dm)
    return logp[:, :NLABEL]


# speedup vs baseline: 1.2350x; 1.2350x over previous
import functools

import jax
import jax.numpy as jnp
import numpy as np
from jax import lax
from jax.experimental import pallas as pl
from jax.experimental.pallas import tpu as pltpu
from jax.experimental.pallas import tpu_sc as plsc

N = 10000
E = 320000
HEADS = 8
CDIM = 16
NH = HEADS * CDIM
NLABEL = 40

_SS = np.zeros((NH, 16), np.float32)
for _h in range(HEADS):
    _SS[_h * CDIM:(_h + 1) * CDIM, _h] = 1.0
    _SS[_h * CDIM:(_h + 1) * CDIM, 8 + _h] = 1.0
_D = np.zeros((16, NH), np.float32)
for _h in range(HEADS):
    _D[_h, _h * CDIM:(_h + 1) * CDIM] = 1.0

_RB = 2000
_GRID = N // _RB

_NC = 2
_NS = 16
_NW = _NC * _NS
_EPW = E // _NW
_K = 16
_NCH = _EPW // _K
_NPAIR = _NCH // 2
_IR = (_EPW + 1279) // 1280 * 10
_NACC = 10240
_NWR = _NACC // 8
_TROWS = _NACC + _NWR
_RPT = _TROWS // _NS


def _make_sc_edge_pass():
    mesh = plsc.VectorSubcoreMesh(
        core_axis_name="c", subcore_axis_name="s",
        num_cores=_NC, num_subcores=_NS)

    @functools.partial(
        pl.kernel,
        out_type=jax.ShapeDtypeStruct((_NC * _TROWS, NH), jnp.float32),
        mesh=mesh,
        scratch_types=[
            pltpu.VMEM((_IR, 128), jnp.int32),
            pltpu.VMEM((_K, 2 * NH), jnp.float32),
            pltpu.VMEM((_K, 2 * NH), jnp.float32),
            pltpu.VMEM((_K, 2 * NH), jnp.float32),
            pltpu.VMEM((_K, NH), jnp.float32),
            pltpu.VMEM((_K, NH), jnp.float32),
            pltpu.VMEM((_K, NH), jnp.float32),
            pltpu.VMEM((2 * _K, NH), jnp.float32),
            pltpu.VMEM((2 * _K,), jnp.int32),
            pltpu.VMEM((16,), jnp.int32),
            pltpu.VMEM_SHARED((_TROWS, NH), jnp.float32),
            pltpu.SemaphoreType.DMA,
            pltpu.SemaphoreType.DMA,
            pltpu.SemaphoreType.DMA,
            pltpu.SemaphoreType.DMA,
        ],
    )
    def sc_edge_pass(hfts_hbm, td_hbm, sd_hbm, zer_hbm, out_hbm,
                     sd_i, hb0, hb1, hb2, tb0, tb1, tb2,
                     cb, ci, offp, acc_sh,
                     gsem0, gsem1, gsem2, ssem):
        cid = lax.axis_index("c")
        sid = lax.axis_index("s")
        wid = sid * _NC + cid

        pltpu.sync_copy(zer_hbm, acc_sh.at[pl.ds(sid * _RPT, _RPT)])
        plsc.subcore_barrier()

        pltpu.sync_copy(sd_hbm.at[wid], sd_i)

        hb = (hb0, hb1, hb2)
        tb = (tb0, tb1, tb2)
        gsem = (gsem0, gsem1, gsem2)
        zeros16 = jnp.zeros((16,), jnp.float32)

        for r in range(_K, 2 * _K):
            for h in range(HEADS):
                cb[r, h * 16:(h + 1) * 16] = zeros16
        offp[...] = jnp.zeros((16,), jnp.int32)

        def idx16(j):
            sd = sd_i[lax.shift_right_logical(j, 3),
                      pl.ds((j & 7) * 16, 16)]
            return sd & 0xFFFF, lax.shift_right_logical(sd, 16)

        def issue(j, slot):
            sv, dv = idx16(j)
            pltpu.async_copy(hfts_hbm.at[sv], hb[slot], gsem[slot])
            pltpu.async_copy(td_hbm.at[dv], tb[slot], gsem[slot])

        def wait2(slot):
            pltpu.make_async_copy(
                hfts_hbm.at[pl.ds(0, _K)], hb[slot], gsem[slot]).wait()
            pltpu.make_async_copy(
                td_hbm.at[pl.ds(0, _K)], tb[slot], gsem[slot]).wait()

        def s_drain():
            pltpu.make_async_copy(
                zer_hbm.at[pl.ds(0, 2 * _K)], cb, ssem).wait()

        def process(j, slot):
            s_drain()
            bh = hb[slot]
            bt = tb[slot]
            _, dv = idx16(j)
            ci[pl.ds(0, 16)] = dv
            ci[pl.ds(16, 16)] = _NACC + lax.shift_right_logical(dv, 3)
            offs = (dv & 7) * 16
            ov = offp[...]
            for j2 in range(16):
                w = bh[j2, 128:144] + bt[j2, 0:16]
                w = jnp.exp(jnp.where(w >= 0.0, w, 0.2 * w))
                for h in range(HEADS):
                    cb[j2, h * 16:(h + 1) * 16] = (
                        bh[j2, h * 16:(h + 1) * 16] * w[h])
                cb[_K + j2, pl.ds(ov[j2], 16)] = zeros16
                cb[_K + j2, pl.ds(offs[j2], 16)] = w
            offp[...] = offs
            pltpu.async_copy(cb, acc_sh.at[ci], ssem, add=True)

        ci[pl.ds(0, 16)] = jnp.full((16,), _NACC - 8, jnp.int32)
        ci[pl.ds(16, 16)] = jnp.full((16,), _TROWS - 8, jnp.int32)
        pltpu.async_copy(cb, acc_sh.at[ci], ssem, add=True)

        issue(0, 0)
        issue(1, 1)

        def tri_body(q, carry):
            j0 = q * 3
            for c in range(3):
                wait2(c)
                issue(j0 + c + 2, (c + 2) % 3)
                process(j0 + c, c)
            return carry

        lax.fori_loop(0, (_NCH - 1) // 3, tri_body, 0)
        wait2(0)
        process(_NCH - 1, 0)
        wait2(1)
        s_drain()

        plsc.subcore_barrier()
        pltpu.sync_copy(
            acc_sh.at[pl.ds(sid * _RPT, _RPT)],
            out_hbm.at[pl.ds(cid * _TROWS + sid * _RPT, _RPT)])

    return sc_edge_pass


_sc_edge_cached = None


def _sc_edge_pass(hfts, td, sd, zer):
    global _sc_edge_cached
    if _sc_edge_cached is None:
        _sc_edge_cached = _make_sc_edge_pass()
    return _sc_edge_cached(hfts, td, sd, zer)


def _wblock(shape):
    nd = len(shape)
    return pl.BlockSpec(shape, lambda i: (0,) * nd)


def _rblock(lanes):
    return pl.BlockSpec((_RB, lanes), lambda i: (i, 0))


def _layer_tail(hf, aS, aD, Dm):
    ts = jnp.dot(hf, aS, preferred_element_type=jnp.float32)
    td = jnp.dot(hf, aD, preferred_element_type=jnp.float32)
    es = ts[:, :16] + td[:, :16]
    ws = jnp.exp(jnp.where(es >= 0.0, es, 0.2 * es))
    sm = hf * jnp.dot(ws, Dm, preferred_element_type=jnp.float32)
    hfts = jnp.concatenate([hf, ts], axis=-1)
    return hfts, td, sm, ws


_TC_PRE_IN = [
    _rblock(128), _wblock((128, 128)), _wblock((128, 128)),
    _wblock((128, 128)), _wblock((16, 128))]
_TC5_OUT = [_rblock(256), _rblock(128), _rblock(128), _rblock(16)]
_TC5_SHAPE = [jax.ShapeDtypeStruct((N, 256), jnp.float32),
              jax.ShapeDtypeStruct((N, 128), jnp.float32),
              jax.ShapeDtypeStruct((N, 128), jnp.float32),
              jax.ShapeDtypeStruct((N, 16), jnp.float32)]


def _tc_pre_body(x_ref, w_ref, aS_ref, aD_ref, D_ref,
                 hfts_ref, td_ref, sm_ref, sw_ref):
    hf = jnp.dot(x_ref[...], w_ref[...], preferred_element_type=jnp.float32)
    hfts, td, sm, ws = _layer_tail(hf, aS_ref[...], aD_ref[...], D_ref[...])
    hfts_ref[...] = hfts
    td_ref[...] = td
    sm_ref[...] = sm
    sw_ref[...] = ws


_tc_pre = pl.pallas_call(
    _tc_pre_body,
    grid=(_GRID,),
    in_specs=_TC_PRE_IN,
    out_specs=_TC5_OUT,
    out_shape=_TC5_SHAPE,
)


def _combine(a0m_ref, a1m_ref, a0w_ref, a1w_ref, sm_ref, sw_ref, b_ref, Dm):
    out = a0m_ref[...] + a1m_ref[...] + sm_ref[...]
    den = a0w_ref[...] + a1w_ref[...] + sw_ref[...]
    dend = jnp.dot(den, Dm, preferred_element_type=jnp.float32) + 1e-16
    z = out / dend + b_ref[...]
    return jnp.where(z > 0.0, z, jnp.exp(jnp.minimum(z, 0.0)) - 1.0)


def _tc_mid_body(a0m_ref, a1m_ref, a0w_ref, a1w_ref, sm0_ref, sw0_ref,
                 b_ref, w_ref, aS_ref, aD_ref, D_ref,
                 hfts_ref, td_ref, sm_ref, sw_ref):
    x1 = _combine(a0m_ref, a1m_ref, a0w_ref, a1w_ref, sm0_ref, sw0_ref,
                  b_ref, D_ref[...])
    hf = jnp.dot(x1, w_ref[...], preferred_element_type=jnp.float32)
    hfts, td, sm, ws = _layer_tail(hf, aS_ref[...], aD_ref[...], D_ref[...])
    hfts_ref[...] = hfts
    td_ref[...] = td
    sm_ref[...] = sm
    sw_ref[...] = ws


def _acc_blocks():
    return [_rblock(128), _rblock(128), _rblock(16), _rblock(16)]


_TC_MID_IN = _acc_blocks() + [
    _rblock(128), _rblock(16), _wblock((1, 128)), _wblock((128, 128)),
    _wblock((128, 128)), _wblock((128, 128)), _wblock((16, 128))]
_TC_FIN_IN = _acc_blocks() + [
    _rblock(128), _rblock(16), _wblock((1, 128)), _wblock((128, 256)),
    _wblock((1, 256)), _wblock((256, 128)), _wblock((1, 128)),
    _wblock((16, 128))]

_tc_mid = pl.pallas_call(
    _tc_mid_body,
    grid=(_GRID,),
    in_specs=_TC_MID_IN,
    out_specs=_TC5_OUT,
    out_shape=_TC5_SHAPE,
)


def _tc_fin_body(a0m_ref, a1m_ref, a0w_ref, a1w_ref, sm1_ref, sw1_ref,
                 b_ref, wf1_ref, bf1_ref, wf2_ref, bf2_ref, D_ref, out_ref):
    x2 = _combine(a0m_ref, a1m_ref, a0w_ref, a1w_ref, sm1_ref, sw1_ref,
                  b_ref, D_ref[...])
    z = jnp.dot(x2, wf1_ref[...], preferred_element_type=jnp.float32)
    z = z + bf1_ref[...]
    z = jnp.where(z > 0.0, z, jnp.exp(jnp.minimum(z, 0.0)) - 1.0)
    z = jnp.dot(z, wf2_ref[...], preferred_element_type=jnp.float32)
    z = z + bf2_ref[...]
    m = jnp.max(z, axis=-1, keepdims=True)
    lse = jnp.log(jnp.sum(jnp.exp(z - m), axis=-1, keepdims=True)) + m
    out_ref[...] = z - lse


_tc_fin = pl.pallas_call(
    _tc_fin_body,
    grid=(_GRID,),
    in_specs=_TC_FIN_IN,
    out_specs=_rblock(128),
    out_shape=jax.ShapeDtypeStruct((N, 128), jnp.float32),
)


def kernel(x, adj, W0, att_src0, att_dst0, b0, W1, att_src1, att_dst1, b1,
           Wf1, bf1, Wf2, bf2):
    SS = jnp.asarray(_SS)
    Dm = jnp.asarray(_D)
    pad = ((0, 0), (0, NH - 16))
    aS0 = jnp.pad(att_src0.reshape(NH)[:, None] * SS, pad)
    aD0 = jnp.pad(att_dst0.reshape(NH)[:, None] * SS, pad)
    aS1 = jnp.pad(att_src1.reshape(NH)[:, None] * SS, pad)
    aD1 = jnp.pad(att_dst1.reshape(NH)[:, None] * SS, pad)
    idx_pad = ((0, 0), (0, _IR * 128 - _EPW))
    sdp = (adj[1] << 16) | adj[0]
    sd = jnp.pad(sdp.reshape(_NW, _EPW), idx_pad).reshape(_NW, _IR, 128)
    zer = jnp.zeros((_RPT, NH), jnp.float32)
    wf2p = jnp.concatenate(
        [Wf2, jnp.zeros((2 * NH, 128 - NLABEL), jnp.float32)], axis=1)
    bf2p = jnp.concatenate(
        [bf2, jnp.full((128 - NLABEL,), -1e30, jnp.float32)]).reshape(1, 128)

    def acc_slices(acc):
        m0 = acc[:N]
        w0 = acc[_NACC:_TROWS].reshape(_NACC, 16)[:N]
        m1 = acc[_TROWS:_TROWS + N]
        w1 = acc[_TROWS + _NACC:2 * _TROWS].reshape(_NACC, 16)[:N]
        return m0, m1, w0, w1

    hfts0, td0, sm0, sw0 = _tc_pre(x, W0, aS0, aD0, Dm)
    acc0 = _sc_edge_pass(hfts0, td0, sd, zer)
    hfts1, td1, sm1, sw1 = _tc_mid(
        *acc_slices(acc0), sm0, sw0, b0.reshape(1, NH), W1, aS1, aD1, Dm)
    acc1 = _sc_edge_pass(hfts1, td1, sd, zer)
    logp = _tc_fin(
        *acc_slices(acc1), sm1, sw1, b1.reshape(1, NH), Wf1,
        bf1.reshape(1, 2 * NH), wf2p, bf2p, Dm)
    return logp[:, :NLABEL]
